# baseline (speedup 1.0000x reference)
import functools

import jax
import jax.numpy as jnp
from jax import lax
from jax.experimental import pallas as pl
from jax.experimental.pallas import tpu as pltpu
from jax.experimental.pallas import tpu_sc as plsc

_N = 10000
_E = 320000
_D = 128
_C = 40
_NC, _NS = 2, 16
_NW = _NC * _NS
_K = 128
_GRP = 16
_NG = 5
_CH = _GRP * _NG
_EPW = _CH * _K
_NPAD = 10112
_RPS = _NPAD // _NS
_DUMMY = _N

_mesh = plsc.VectorSubcoreMesh(
    core_axis_name="c", subcore_axis_name="s", num_cores=_NC, num_subcores=_NS
)


def _agg_body(h_hbm, src_hbm, dst_hbm, zeros_hbm, acc_hbm,
              sidx, didx, b0, b1, acc_sh, s0, s1):
    c = lax.axis_index("c")
    s = lax.axis_index("s")
    rows = pl.ds(s * _RPS, _RPS)
    bufs = (b0, b1)
    sems = (s0, s1)
    pltpu.sync_copy(zeros_hbm.at[rows], acc_sh.at[rows])
    plsc.subcore_barrier()

    def group(g, carry):
        pltpu.sync_copy(src_hbm.at[c, s, pl.ds(g * _GRP, _GRP)], sidx)
        pltpu.sync_copy(dst_hbm.at[c, s, pl.ds(g * _GRP, _GRP)], didx)
        handles = [
            pltpu.async_copy(h_hbm.at[sidx.at[b]], bufs[b], sems[b])
            for b in range(2)
        ]
        for j in range(_GRP):
            b = j % 2
            handles[b].wait()
            pltpu.sync_copy(bufs[b], acc_sh.at[didx.at[j]], add=True)
            if j + 2 < _GRP:
                handles[b] = pltpu.async_copy(
                    h_hbm.at[sidx.at[j + 2]], bufs[b], sems[b])
        return carry

    lax.fori_loop(0, _NG, group, 0)
    plsc.subcore_barrier()
    pltpu.sync_copy(acc_sh.at[rows], acc_hbm.at[c, rows])


_agg_call = functools.partial(
    pl.kernel,
    _agg_body,
    out_type=jax.ShapeDtypeStruct((_NC, _NPAD, _D), jnp.float32),
    mesh=_mesh,
    scratch_types=[
        pltpu.VMEM((_GRP, _K), jnp.int32),
        pltpu.VMEM((_GRP, _K), jnp.int32),
        pltpu.VMEM((_K, _D), jnp.float32),
        pltpu.VMEM((_K, _D), jnp.float32),
        pltpu.VMEM_SHARED((_NPAD, _D), jnp.float32),
        pltpu.SemaphoreType.DMA,
        pltpu.SemaphoreType.DMA,
    ],
)()


_DW = 16


def _deg_body(dst_hbm, ones_hbm, zeros_hbm, deg_hbm, dst_v, ones_v, deg_sh):
    c = lax.axis_index("c")
    s = lax.axis_index("s")
    rows = pl.ds(s * _RPS, _RPS)
    pltpu.sync_copy(dst_hbm.at[c, s], dst_v)
    pltpu.sync_copy(ones_hbm, ones_v)
    pltpu.sync_copy(zeros_hbm.at[rows], deg_sh.at[rows])
    plsc.subcore_barrier()

    def body(j, carry):
        pltpu.sync_copy(ones_v, deg_sh.at[dst_v.at[j]], add=True)
        return carry

    lax.fori_loop(0, _CH, body, 0)
    plsc.subcore_barrier()
    pltpu.sync_copy(deg_sh.at[rows], deg_hbm.at[c, rows])


_deg_call = functools.partial(
    pl.kernel,
    _deg_body,
    out_type=jax.ShapeDtypeStruct((_NC, _NPAD, _D), jnp.float32),
    mesh=_mesh,
    scratch_types=[
        pltpu.VMEM((_CH, _K), jnp.int32),
        pltpu.VMEM((_K, _D), jnp.float32),
        pltpu.VMEM_SHARED((_NPAD, _D), jnp.float32),
    ],
)()


_BM = 1000


def _layer_body(acc_ref, deg_ref, h_ref, wl_ref, wr_ref, b_ref, out_ref, *, act):
    a = acc_ref[0] + acc_ref[1]
    dg = deg_ref[0, :, 0:1] + deg_ref[1, :, 0:1]
    mean = a * (1.0 / jnp.maximum(dg, 1.0))
    z = (jnp.dot(mean, wl_ref[...], preferred_element_type=jnp.float32)
         + jnp.dot(h_ref[...], wr_ref[...], preferred_element_type=jnp.float32)
         + b_ref[...])
    if act:
        z = jnp.where(z > 0, z, jnp.exp(z) - 1.0)
    out_ref[...] = z


def _layer_call(acc, deg, h, wl, wr, bias, act):
    return pl.pallas_call(
        functools.partial(_layer_body, act=act),
        grid=(_N // _BM,),
        in_specs=[
            pl.BlockSpec((_NC, _BM, _D), lambda i: (0, i, 0)),
            pl.BlockSpec((_NC, _BM, _D), lambda i: (0, i, 0)),
            pl.BlockSpec((_BM, _D), lambda i: (i, 0)),
            pl.BlockSpec((_D, _D), lambda i: (0, 0)),
            pl.BlockSpec((_D, _D), lambda i: (0, 0)),
            pl.BlockSpec((1, _D), lambda i: (0, 0)),
        ],
        out_specs=pl.BlockSpec((_BM, _D), lambda i: (i, 0)),
        out_shape=jax.ShapeDtypeStruct((_N, _D), jnp.float32),
    )(acc, deg, h, wl, wr, bias)


def kernel(x, edge_index, weight, Wl, Wr, b, Wl9, Wr9, b9):
    del weight
    src = edge_index[0].astype(jnp.int32)
    dst = edge_index[1].astype(jnp.int32)
    pad = _NW * _EPW - _E
    src_p = jnp.concatenate([src, jnp.zeros((pad,), jnp.int32)])
    src_p = src_p.reshape(_NC, _NS, _CH, _K)
    dst_p = jnp.concatenate([dst, jnp.full((pad,), _DUMMY, jnp.int32)])
    dst_p = dst_p.reshape(_NC, _NS, _CH, _K)

    zeros128 = jnp.zeros((_NPAD, _D), jnp.float32)
    ones128 = jnp.ones((_K, _D), jnp.float32)

    deg = _deg_call(dst_p, ones128, zeros128)

    h = x.astype(jnp.float32)
    for i in range(8):
        acc = _agg_call(h, src_p, dst_p, zeros128)
        h = _layer_call(acc, deg, h, Wl[i], Wr[i], b[i][None, :], act=True)

    acc = _agg_call(h, src_p, dst_p, zeros128)
    wl9 = jnp.pad(Wl9, ((0, 0), (0, _D - _C)))
    wr9 = jnp.pad(Wr9, ((0, 0), (0, _D - _C)))
    b9p = jnp.pad(b9, (0, _D - _C))[None, :]
    out = _layer_call(acc, deg, h, wl9, wr9, b9p, act=False)
    return out[:, :_C]


# speedup vs baseline: 1.0054x; 1.0054x over previous
import functools

import jax
import jax.numpy as jnp
from jax import lax
from jax.experimental import pallas as pl
from jax.experimental.pallas import tpu as pltpu
from jax.experimental.pallas import tpu_sc as plsc

_N = 10000
_E = 320000
_D = 128
_C = 40
_NC, _NS = 2, 16
_NW = _NC * _NS
_K = 128
_GRP = 16
_NG = 5
_CH = _GRP * _NG
_EPW = _CH * _K
_NPAD = 10112
_RPS = _NPAD // _NS
_DUMMY = _N

_mesh = plsc.VectorSubcoreMesh(
    core_axis_name="c", subcore_axis_name="s", num_cores=_NC, num_subcores=_NS
)


def _agg_body(h_hbm, src_hbm, dst_hbm, zeros_hbm, acc_hbm,
              sidx, didx, b0, b1, acc_sh, s0, s1, si):
    c = lax.axis_index("c")
    s = lax.axis_index("s")
    rows = pl.ds(s * _RPS, _RPS)
    bufs = (b0, b1)
    sems = (s0, s1)
    pltpu.sync_copy(zeros_hbm.at[rows], acc_sh.at[rows])
    pltpu.sync_copy(src_hbm.at[c, s, pl.ds(0, _GRP)], sidx.at[pl.ds(0, _GRP)])
    pltpu.sync_copy(dst_hbm.at[c, s, pl.ds(0, _GRP)], didx.at[pl.ds(0, _GRP)])
    plsc.subcore_barrier()

    def group(g, carry):
        p = lax.rem(g, 2)
        pn = lax.rem(g + 1, 2)
        gn = jnp.minimum(g + 1, _NG - 1)
        pbase = p * _GRP
        pltpu.async_copy(src_hbm.at[c, s, pl.ds(gn * _GRP, _GRP)],
                         sidx.at[pl.ds(pn * _GRP, _GRP)], si)
        pltpu.async_copy(dst_hbm.at[c, s, pl.ds(gn * _GRP, _GRP)],
                         didx.at[pl.ds(pn * _GRP, _GRP)], si)
        handles = [
            pltpu.async_copy(h_hbm.at[sidx.at[pbase + b]], bufs[b], sems[b])
            for b in range(2)
        ]
        for j in range(_GRP):
            b = j % 2
            handles[b].wait()
            pltpu.sync_copy(bufs[b], acc_sh.at[didx.at[pbase + j]], add=True)
            if j + 2 < _GRP:
                handles[b] = pltpu.async_copy(
                    h_hbm.at[sidx.at[pbase + j + 2]], bufs[b], sems[b])
        pltpu.make_async_copy(src_hbm.at[c, s, pl.ds(gn * _GRP, _GRP)],
                              sidx.at[pl.ds(pn * _GRP, _GRP)], si).wait()
        pltpu.make_async_copy(dst_hbm.at[c, s, pl.ds(gn * _GRP, _GRP)],
                              didx.at[pl.ds(pn * _GRP, _GRP)], si).wait()
        return carry

    lax.fori_loop(0, _NG, group, 0)
    plsc.subcore_barrier()
    pltpu.sync_copy(acc_sh.at[rows], acc_hbm.at[c, rows])


_agg_call = functools.partial(
    pl.kernel,
    _agg_body,
    out_type=jax.ShapeDtypeStruct((_NC, _NPAD, _D), jnp.float32),
    mesh=_mesh,
    scratch_types=[
        pltpu.VMEM((2 * _GRP, _K), jnp.int32),
        pltpu.VMEM((2 * _GRP, _K), jnp.int32),
        pltpu.VMEM((_K, _D), jnp.float32),
        pltpu.VMEM((_K, _D), jnp.float32),
        pltpu.VMEM_SHARED((_NPAD, _D), jnp.float32),
        pltpu.SemaphoreType.DMA,
        pltpu.SemaphoreType.DMA,
        pltpu.SemaphoreType.DMA,
    ],
)()


_DW = 16


def _deg_body(dst_hbm, ones_hbm, zeros_hbm, deg_hbm, dst_v, ones_v, deg_sh):
    c = lax.axis_index("c")
    s = lax.axis_index("s")
    rows = pl.ds(s * _RPS, _RPS)
    pltpu.sync_copy(dst_hbm.at[c, s], dst_v)
    pltpu.sync_copy(ones_hbm, ones_v)
    pltpu.sync_copy(zeros_hbm.at[rows], deg_sh.at[rows])
    plsc.subcore_barrier()

    def body(j, carry):
        pltpu.sync_copy(ones_v, deg_sh.at[dst_v.at[j]], add=True)
        return carry

    lax.fori_loop(0, _CH, body, 0)
    plsc.subcore_barrier()
    pltpu.sync_copy(deg_sh.at[rows], deg_hbm.at[c, rows])


_deg_call = functools.partial(
    pl.kernel,
    _deg_body,
    out_type=jax.ShapeDtypeStruct((_NC, _NPAD, _D), jnp.float32),
    mesh=_mesh,
    scratch_types=[
        pltpu.VMEM((_CH, _K), jnp.int32),
        pltpu.VMEM((_K, _D), jnp.float32),
        pltpu.VMEM_SHARED((_NPAD, _D), jnp.float32),
    ],
)()


_BM = 1000


def _layer_body(acc_ref, deg_ref, h_ref, wl_ref, wr_ref, b_ref, out_ref, *, act):
    a = acc_ref[0] + acc_ref[1]
    dg = deg_ref[0, :, 0:1] + deg_ref[1, :, 0:1]
    mean = a * (1.0 / jnp.maximum(dg, 1.0))
    z = (jnp.dot(mean, wl_ref[...], preferred_element_type=jnp.float32)
         + jnp.dot(h_ref[...], wr_ref[...], preferred_element_type=jnp.float32)
         + b_ref[...])
    if act:
        z = jnp.where(z > 0, z, jnp.exp(z) - 1.0)
    out_ref[...] = z


def _layer_call(acc, deg, h, wl, wr, bias, act):
    return pl.pallas_call(
        functools.partial(_layer_body, act=act),
        grid=(_N // _BM,),
        in_specs=[
            pl.BlockSpec((_NC, _BM, _D), lambda i: (0, i, 0)),
            pl.BlockSpec((_NC, _BM, _D), lambda i: (0, i, 0)),
            pl.BlockSpec((_BM, _D), lambda i: (i, 0)),
            pl.BlockSpec((_D, _D), lambda i: (0, 0)),
            pl.BlockSpec((_D, _D), lambda i: (0, 0)),
            pl.BlockSpec((1, _D), lambda i: (0, 0)),
        ],
        out_specs=pl.BlockSpec((_BM, _D), lambda i: (i, 0)),
        out_shape=jax.ShapeDtypeStruct((_N, _D), jnp.float32),
    )(acc, deg, h, wl, wr, bias)


def kernel(x, edge_index, weight, Wl, Wr, b, Wl9, Wr9, b9):
    del weight
    src = edge_index[0].astype(jnp.int32)
    dst = edge_index[1].astype(jnp.int32)
    pad = _NW * _EPW - _E
    src_p = jnp.concatenate([src, jnp.zeros((pad,), jnp.int32)])
    src_p = src_p.reshape(_NC, _NS, _CH, _K)
    dst_p = jnp.concatenate([dst, jnp.full((pad,), _DUMMY, jnp.int32)])
    dst_p = dst_p.reshape(_NC, _NS, _CH, _K)

    zeros128 = jnp.zeros((_NPAD, _D), jnp.float32)
    ones128 = jnp.ones((_K, _D), jnp.float32)

    deg = _deg_call(dst_p, ones128, zeros128)

    h = x.astype(jnp.float32)
    for i in range(8):
        acc = _agg_call(h, src_p, dst_p, zeros128)
        h = _layer_call(acc, deg, h, Wl[i], Wr[i], b[i][None, :], act=True)

    acc = _agg_call(h, src_p, dst_p, zeros128)
    wl9 = jnp.pad(Wl9, ((0, 0), (0, _D - _C)))
    wr9 = jnp.pad(Wr9, ((0, 0), (0, _D - _C)))
    b9p = jnp.pad(b9, (0, _D - _C))[None, :]
    out = _layer_call(acc, deg, h, wl9, wr9, b9p, act=False)
    return out[:, :_C]
